# baseline (speedup 1.0000x reference)
import functools

import jax
import jax.numpy as jnp
from jax.experimental import pallas as pl


def _agg_kernel(len_ref, xa_ref, xb_ref, wq_ref, out_ref, *, t_chunk: int):
    kt = pl.program_id(1)

    @pl.when(kt == 0)
    def _init():
        out_ref[...] = jnp.zeros_like(out_ref)

    xa = xa_ref[0]
    xb = xb_ref[0]
    dh = xa.shape[1]
    wq = wq_ref[...].reshape(1, 2 * dh)
    w = (jnp.sum(xa * wq[:, :dh], axis=1)
         + jnp.sum(xb * wq[:, dh:], axis=1))

    t0 = kt * t_chunk
    t_idx = jax.lax.broadcasted_iota(jnp.int32, (1, t_chunk), 1) + t0
    mask = (t_idx < len_ref[...]).astype(jnp.float32)
    a = mask * w.reshape(1, t_chunk)

    acc_a = jax.lax.dot_general(
        a, xa, (((1,), (0,)), ((), ())),
        preferred_element_type=jnp.float32)
    acc_b = jax.lax.dot_general(
        a, xb, (((1,), (0,)), ((), ())),
        preferred_element_type=jnp.float32)
    out_ref[0, :, :dh] += acc_a
    out_ref[0, :, dh:] += acc_b


def kernel(lengths, nodes_output, Wq):
    B, T, D = nodes_output.shape
    n_len = lengths.shape[0]
    t_chunk = 1024
    dh = D // 2
    lens = jnp.asarray(lengths, dtype=jnp.int32).reshape(n_len, 1)

    grid = (B, T // t_chunk)
    out = pl.pallas_call(
        functools.partial(_agg_kernel, t_chunk=t_chunk),
        grid=grid,
        in_specs=[
            pl.BlockSpec((n_len, 1), lambda b, kt: (0, 0)),
            pl.BlockSpec((1, t_chunk, dh), lambda b, kt: (b, kt, 0)),
            pl.BlockSpec((1, t_chunk, dh), lambda b, kt: (b, kt, 1)),
            pl.BlockSpec((D, 1), lambda b, kt: (0, 0)),
        ],
        out_specs=pl.BlockSpec((1, n_len, D), lambda b, kt: (b, 0, 0)),
        out_shape=jax.ShapeDtypeStruct((B, n_len, D), jnp.float32),
    )(lens, nodes_output, nodes_output, Wq)
    return out.transpose(1, 0, 2).reshape(n_len * B, D)


# speedup vs baseline: 1.0720x; 1.0720x over previous
import functools

import jax
import jax.numpy as jnp
from jax.experimental import pallas as pl


def _agg_kernel(len_ref, xa_ref, xb_ref, wq_ref, out_ref, *, t_chunk: int):
    kt = pl.program_id(1)

    @pl.when(kt == 0)
    def _init():
        out_ref[...] = jnp.zeros_like(out_ref)

    wq_row = wq_ref[...].reshape(1, -1)
    acc = jnp.zeros_like(out_ref[0])
    for half, x_half in enumerate((xa_ref[0], xb_ref[0])):
        w = jnp.sum(x_half * wq_row, axis=1)
        t0 = (kt * 2 + half) * t_chunk
        t_idx = jax.lax.broadcasted_iota(jnp.int32, (1, t_chunk), 1) + t0
        mask = (t_idx < len_ref[...]).astype(jnp.float32)
        a = mask * w.reshape(1, t_chunk)
        acc += jax.lax.dot_general(
            a, x_half, (((1,), (0,)), ((), ())),
            preferred_element_type=jnp.float32)
    out_ref[0] += acc


def kernel(lengths, nodes_output, Wq):
    B, T, D = nodes_output.shape
    n_len = lengths.shape[0]
    t_chunk = 512
    lens = jnp.asarray(lengths, dtype=jnp.int32).reshape(n_len, 1)

    grid = (B, T // (2 * t_chunk))
    out = pl.pallas_call(
        functools.partial(_agg_kernel, t_chunk=t_chunk),
        grid=grid,
        in_specs=[
            pl.BlockSpec((n_len, 1), lambda b, kt: (0, 0)),
            pl.BlockSpec((1, t_chunk, D), lambda b, kt: (b, 2 * kt, 0)),
            pl.BlockSpec((1, t_chunk, D), lambda b, kt: (b, 2 * kt + 1, 0)),
            pl.BlockSpec((D, 1), lambda b, kt: (0, 0)),
        ],
        out_specs=pl.BlockSpec((1, n_len, D), lambda b, kt: (b, 0, 0)),
        out_shape=jax.ShapeDtypeStruct((B, n_len, D), jnp.float32),
    )(lens, nodes_output, nodes_output, Wq)
    return out.transpose(1, 0, 2).reshape(n_len * B, D)
